# baseline (speedup 1.0000x reference)
import functools

import jax
import jax.numpy as jnp
from jax import lax
from jax.experimental import pallas as pl
from jax.experimental.pallas import tpu as pltpu
from jax.experimental.pallas import tpu_sc as plsc

_VOCAB = 64000
_D = 6
_B = 8
_T, _H, _W = 5, 32, 32
_FRAME = _H * _W
_N_KEEP = (_T - 1) * _FRAME
_NC, _NS = 2, 16
_NW = _NC * _NS
_CHUNK = _B * _N_KEEP // _NW


_DP = 8


def _sc_body(tgt_idx, pred_idx, codebook, tgt_out, pred_out,
             idx_t, idx_p, rows_t, rows_p, sem_t, sem_p):
    wid = lax.axis_index("s") * _NC + lax.axis_index("c")
    b = wid // 4
    q = wid % 4
    src0 = _FRAME + q * _CHUNK
    dst0 = q * _CHUNK

    pltpu.sync_copy(tgt_idx.at[b, pl.ds(src0, _CHUNK)], idx_t)
    pltpu.sync_copy(pred_idx.at[b, pl.ds(src0, _CHUNK)], idx_p)
    ct = pltpu.async_copy(codebook.at[idx_t], rows_t, sem_t)
    cp = pltpu.async_copy(codebook.at[idx_p], rows_p, sem_p)
    ct.wait()
    pltpu.sync_copy(rows_t, tgt_out.at[b, pl.ds(dst0, _CHUNK)])
    cp.wait()
    pltpu.sync_copy(rows_p, pred_out.at[b, pl.ds(dst0, _CHUNK)])


_sc_gather = functools.partial(
    pl.kernel,
    out_type=(
        jax.ShapeDtypeStruct((_B, _N_KEEP, _DP), jnp.float32),
        jax.ShapeDtypeStruct((_B, _N_KEEP, _DP), jnp.float32),
    ),
    mesh=plsc.VectorSubcoreMesh(core_axis_name="c", subcore_axis_name="s"),
    scratch_types=[
        pltpu.VMEM((_CHUNK,), jnp.int32),
        pltpu.VMEM((_CHUNK,), jnp.int32),
        pltpu.VMEM((_CHUNK, _DP), jnp.float32),
        pltpu.VMEM((_CHUNK, _DP), jnp.float32),
        pltpu.SemaphoreType.DMA,
        pltpu.SemaphoreType.DMA,
    ],
    compiler_params=pltpu.CompilerParams(use_tc_tiling_on_sc=False),
)(_sc_body)


def kernel(target_indices, pred_indices, codebook):
    cb_pad = jnp.pad(codebook, ((0, 0), (0, _DP - _D)))
    tgt_f32, pred_f32 = _sc_gather(target_indices, pred_indices, cb_pad)
    return (pred_f32[..., :_D].astype(jnp.float16),
            tgt_f32[..., :_D].astype(jnp.float16))
